# Initial kernel scaffold; baseline (speedup 1.0000x reference)
import jax
import jax.numpy as jnp
from jax.experimental import pallas as pl


def kernel(input_ids, word_embeddings, position_embeddings, lin_w, lin_b, ln_gamma, ln_beta):
    raise NotImplementedError("write your pallas kernel here")


# speedup vs baseline: 1.6394x; 1.6394x over previous
import functools

import jax
import jax.numpy as jnp
from jax import lax
from jax.experimental import pallas as pl
from jax.experimental.pallas import tpu as pltpu
from jax.experimental.pallas import tpu_sc as plsc

VOCAB = 100000
HIDDEN = 768
EPS = 1e-12

NC = 2
NS = 16
NW = NC * NS

TOKENS = 8192
B_PER_W = TOKENS // NW
CHUNK = 64
NCHUNK = B_PER_W // CHUNK


def _sc_gather_body(table_hbm, idx_hbm, out_hbm, idx_v, rows0, rows1, sem0, sem1):
    wid = lax.axis_index("s") * NC + lax.axis_index("c")
    base = wid * B_PER_W
    pltpu.sync_copy(idx_hbm.at[wid], idx_v)
    bufs = (rows0, rows1)
    sems = (sem0, sem1)
    handles = [None, None]
    handles[0] = pltpu.async_copy(table_hbm.at[idx_v.at[0]], rows0, sem0)
    for c in range(NCHUNK):
        nxt = c + 1
        if nxt < NCHUNK:
            handles[nxt % 2] = pltpu.async_copy(
                table_hbm.at[idx_v.at[nxt]], bufs[nxt % 2], sems[nxt % 2])
        handles[c % 2].wait()
        pltpu.sync_copy(bufs[c % 2], out_hbm.at[pl.ds(base + c * CHUNK, CHUNK)])


_sc_gather = pl.kernel(
    _sc_gather_body,
    out_type=jax.ShapeDtypeStruct((TOKENS, HIDDEN), jnp.float32),
    mesh=plsc.VectorSubcoreMesh(core_axis_name="c", subcore_axis_name="s"),
    scratch_types=[
        pltpu.VMEM((NCHUNK, CHUNK), jnp.int32),
        pltpu.VMEM((CHUNK, HIDDEN), jnp.float32),
        pltpu.VMEM((CHUNK, HIDDEN), jnp.float32),
        pltpu.SemaphoreType.DMA,
        pltpu.SemaphoreType.DMA,
    ],
    name="sc_embed_gather",
)


ROWS_BLK = 512
POS_BLKS = 2048 // ROWS_BLK


def _tc_dense_body(x_ref, pos_ref, w_ref, b_ref, g_ref, bt_ref, o_ref):
    x = x_ref[...] + pos_ref[...]
    y = lax.dot_general(
        x.astype(jnp.bfloat16), w_ref[...],
        (((1,), (1,)), ((), ())),
        preferred_element_type=jnp.float32,
    )
    y = y + b_ref[...]
    mean = jnp.mean(y, axis=1, keepdims=True)
    yc = y - mean
    var = jnp.mean(yc * yc, axis=1, keepdims=True)
    o_ref[...] = yc * lax.rsqrt(var + EPS) * g_ref[...] + bt_ref[...]


_tc_dense = pl.pallas_call(
    _tc_dense_body,
    grid=(TOKENS // ROWS_BLK,),
    in_specs=[
        pl.BlockSpec((ROWS_BLK, HIDDEN), lambda i: (i, 0)),
        pl.BlockSpec((ROWS_BLK, HIDDEN), lambda i: (i % POS_BLKS, 0)),
        pl.BlockSpec((HIDDEN, HIDDEN), lambda i: (0, 0)),
        pl.BlockSpec((1, HIDDEN), lambda i: (0, 0)),
        pl.BlockSpec((1, HIDDEN), lambda i: (0, 0)),
        pl.BlockSpec((1, HIDDEN), lambda i: (0, 0)),
    ],
    out_specs=pl.BlockSpec((ROWS_BLK, HIDDEN), lambda i: (i, 0)),
    out_shape=jax.ShapeDtypeStruct((TOKENS, HIDDEN), jnp.float32),
    name="tc_add_linear_ln",
)


def kernel(input_ids, word_embeddings, position_embeddings, lin_w, lin_b,
           ln_gamma, ln_beta):
    batch, seq = input_ids.shape
    ids = input_ids.astype(jnp.int32).reshape(NW, NCHUNK, CHUNK)
    gathered = _sc_gather(word_embeddings, ids)
    out = _tc_dense(
        gathered,
        position_embeddings,
        lin_w.astype(jnp.bfloat16),
        lin_b.reshape(1, HIDDEN),
        ln_gamma.reshape(1, HIDDEN),
        ln_beta.reshape(1, HIDDEN),
    )
    return out.reshape(batch, seq, HIDDEN)


# speedup vs baseline: 1.7973x; 1.0963x over previous
import functools

import jax
import jax.numpy as jnp
from jax import lax
from jax.experimental import pallas as pl
from jax.experimental.pallas import tpu as pltpu
from jax.experimental.pallas import tpu_sc as plsc

VOCAB = 100000
HIDDEN = 768
EPS = 1e-12

NC = 2
NS = 16
NW = NC * NS

TOKENS = 8192
B_PER_W = TOKENS // NW
CHUNK = 64
NCHUNK = B_PER_W // CHUNK


def _sc_gather_body(table_hbm, idx_hbm, out_hbm, idx_v, rows0, rows1, sem0, sem1):
    wid = lax.axis_index("s") * NC + lax.axis_index("c")
    base = wid * B_PER_W
    pltpu.sync_copy(idx_hbm.at[wid], idx_v)
    bufs = (rows0, rows1)
    sems = (sem0, sem1)
    handles = [None, None]
    handles[0] = pltpu.async_copy(table_hbm.at[idx_v.at[0]], rows0, sem0)
    for c in range(NCHUNK):
        nxt = c + 1
        if nxt < NCHUNK:
            handles[nxt % 2] = pltpu.async_copy(
                table_hbm.at[idx_v.at[nxt]], bufs[nxt % 2], sems[nxt % 2])
        handles[c % 2].wait()
        pltpu.sync_copy(bufs[c % 2], out_hbm.at[pl.ds(base + c * CHUNK, CHUNK)])


_sc_gather = pl.kernel(
    _sc_gather_body,
    out_type=jax.ShapeDtypeStruct((TOKENS, HIDDEN), jnp.float32),
    mesh=plsc.VectorSubcoreMesh(core_axis_name="c", subcore_axis_name="s"),
    scratch_types=[
        pltpu.VMEM((NCHUNK, CHUNK), jnp.int32),
        pltpu.VMEM((CHUNK, HIDDEN), jnp.float32),
        pltpu.VMEM((CHUNK, HIDDEN), jnp.float32),
        pltpu.SemaphoreType.DMA,
        pltpu.SemaphoreType.DMA,
    ],
    name="sc_embed_gather",
)


ROWS_BLK = 1024
POS_BLKS = 2048 // ROWS_BLK
BATCHES = TOKENS // 2048


def _tc_dense_body(x_ref, pos_ref, w_ref, b_ref, g_ref, bt_ref, o_ref):
    x = x_ref[...] + pos_ref[...]
    y = lax.dot_general(
        x.astype(jnp.bfloat16), w_ref[...],
        (((1,), (1,)), ((), ())),
        preferred_element_type=jnp.float32,
    )
    y = y + b_ref[...]
    mean = jnp.mean(y, axis=1, keepdims=True)
    yc = y - mean
    var = jnp.mean(yc * yc, axis=1, keepdims=True)
    o_ref[...] = yc * lax.rsqrt(var + EPS) * g_ref[...] + bt_ref[...]


_tc_dense = pl.pallas_call(
    _tc_dense_body,
    grid=(POS_BLKS, BATCHES),
    in_specs=[
        pl.BlockSpec((ROWS_BLK, HIDDEN), lambda pb, b: (b * POS_BLKS + pb, 0)),
        pl.BlockSpec((ROWS_BLK, HIDDEN), lambda pb, b: (pb, 0)),
        pl.BlockSpec((HIDDEN, HIDDEN), lambda pb, b: (0, 0)),
        pl.BlockSpec((1, HIDDEN), lambda pb, b: (0, 0)),
        pl.BlockSpec((1, HIDDEN), lambda pb, b: (0, 0)),
        pl.BlockSpec((1, HIDDEN), lambda pb, b: (0, 0)),
    ],
    out_specs=pl.BlockSpec((ROWS_BLK, HIDDEN), lambda pb, b: (b * POS_BLKS + pb, 0)),
    out_shape=jax.ShapeDtypeStruct((TOKENS, HIDDEN), jnp.float32),
    name="tc_add_linear_ln",
)


def kernel(input_ids, word_embeddings, position_embeddings, lin_w, lin_b,
           ln_gamma, ln_beta):
    batch, seq = input_ids.shape
    ids = input_ids.astype(jnp.int32).reshape(NW, NCHUNK, CHUNK)
    gathered = _sc_gather(word_embeddings, ids)
    out = _tc_dense(
        gathered,
        position_embeddings,
        lin_w.astype(jnp.bfloat16),
        lin_b.reshape(1, HIDDEN),
        ln_gamma.reshape(1, HIDDEN),
        ln_beta.reshape(1, HIDDEN),
    )
    return out.reshape(batch, seq, HIDDEN)
